# baseline (speedup 1.0000x reference)
import functools

import jax
import jax.numpy as jnp
from jax import lax
from jax.experimental import pallas as pl
from jax.experimental.pallas import tpu as pltpu
from jax.experimental.pallas import tpu_sc as plsc

_BATCH = 4096
_SEQ = 50
_D = 64
_B = _BATCH * _SEQ
_NW = 32
_BPW = _B // _NW
_CH = 128
_NCH = _BPW // _CH


def _build():
  mesh = plsc.VectorSubcoreMesh(core_axis_name="c", subcore_axis_name="s")

  @functools.partial(
      pl.kernel,
      mesh=mesh,
      compiler_params=pltpu.CompilerParams(use_tc_tiling_on_sc=False),
      out_type=jax.ShapeDtypeStruct((_B, _D), jnp.float32),
      scratch_types=[
          pltpu.VMEM((_NCH, _CH), jnp.int32),
          pltpu.VMEM((_CH, _D), jnp.float32),
          pltpu.SemaphoreType.DMA,
      ],
  )
  def k(idx_hbm, table_hbm, out_hbm, idx_v, buf, gsem):
    wid = lax.axis_index("s") * 2 + lax.axis_index("c")
    base = wid * _BPW
    pltpu.sync_copy(idx_hbm.at[wid], idx_v)

    def body(j, _):
      pltpu.async_copy(table_hbm.at[idx_v.at[j]], buf, gsem).wait()
      pltpu.sync_copy(buf, out_hbm.at[pl.ds(base + j * _CH, _CH)])
      return 0

    lax.fori_loop(0, _NCH, body, 0)

  return k


_gather_kernel = _build()


def kernel(tok_idxs, embed):
  idx = tok_idxs.reshape(_NW, _NCH, _CH)
  out = _gather_kernel(idx, embed)
  return out.reshape(_BATCH, _SEQ, _D)


# speedup vs baseline: 1.4624x; 1.4624x over previous
import functools

import jax
import jax.numpy as jnp
from jax import lax
from jax.experimental import pallas as pl
from jax.experimental.pallas import tpu as pltpu
from jax.experimental.pallas import tpu_sc as plsc

_BATCH = 4096
_SEQ = 50
_D = 64
_B = _BATCH * _SEQ
_NW = 32
_BPW = _B // _NW
_NB = _BATCH // _NW
_CBATCH = 2
_CH = _CBATCH * _SEQ
_NCH = _NB // _CBATCH


def _build():
  mesh = plsc.VectorSubcoreMesh(core_axis_name="c", subcore_axis_name="s")

  @functools.partial(
      pl.kernel,
      mesh=mesh,
      out_type=jax.ShapeDtypeStruct((_BATCH, _SEQ, _D), jnp.float32),
      scratch_types=[
          pltpu.VMEM((_BPW,), jnp.int32),
          pltpu.VMEM((_CBATCH, _SEQ, _D), jnp.float32),
          pltpu.SemaphoreType.DMA,
          pltpu.SemaphoreType.DMA,
      ],
  )
  def k(idx_hbm, table_hbm, out_hbm, idx_v, buf, isem, gsem):
    wid = lax.axis_index("s") * 2 + lax.axis_index("c")
    pltpu.async_copy(idx_hbm.at[wid], idx_v, isem).wait()

    def chunk_body(j, _):
      base = j * _CH

      def grp_body(g, _):
        vec = idx_v[pl.ds(base + g * 16, 16)]
        for lane in range(16):
          i = g * 16 + lane
          r = vec[lane]
          pltpu.async_copy(
              table_hbm.at[pl.ds(r, 1)],
              buf.at[i // _SEQ, pl.ds(i % _SEQ, 1)],
              gsem,
          )
        return 0

      lax.fori_loop(0, _CH // 16, grp_body, 0, unroll=True)
      vec = idx_v[pl.ds(base + 96, 16)]
      for lane in range(_CH - 96):
        i = 96 + lane
        r = vec[lane]
        pltpu.async_copy(
            table_hbm.at[pl.ds(r, 1)],
            buf.at[i // _SEQ, pl.ds(i % _SEQ, 1)],
            gsem,
        )
      dst = out_hbm.at[pl.ds(wid * _NB + j * _CBATCH, _CBATCH)]
      pltpu.make_async_copy(dst, buf, gsem).wait()
      pltpu.sync_copy(buf, dst)
      return 0

    lax.fori_loop(0, _NCH, chunk_body, 0)

  return k


_gather_kernel = _build()


def kernel(tok_idxs, embed):
  idx = tok_idxs.reshape(_NW, _BPW)
  return _gather_kernel(idx, embed)


# speedup vs baseline: 1.5960x; 1.0913x over previous
import functools

import jax
import jax.numpy as jnp
from jax import lax
from jax.experimental import pallas as pl
from jax.experimental.pallas import tpu as pltpu
from jax.experimental.pallas import tpu_sc as plsc

_BATCH = 4096
_SEQ = 50
_D = 64
_B = _BATCH * _SEQ
_NW = 32
_BPW = _B // _NW
_NB = _BATCH // _NW
_CBATCH = 4
_CH = _CBATCH * _SEQ
_NCH = _NB // _CBATCH


def _build():
  mesh = plsc.VectorSubcoreMesh(core_axis_name="c", subcore_axis_name="s")

  @functools.partial(
      pl.kernel,
      mesh=mesh,
      out_type=jax.ShapeDtypeStruct((_BATCH, _SEQ, _D), jnp.float32),
      scratch_types=[
          pltpu.VMEM((_BPW,), jnp.int32),
          pltpu.VMEM((2, _CBATCH, _SEQ, _D), jnp.float32),
          pltpu.SemaphoreType.DMA,
          pltpu.SemaphoreType.DMA,
          pltpu.SemaphoreType.DMA,
          pltpu.SemaphoreType.DMA,
          pltpu.SemaphoreType.DMA,
      ],
  )
  def k(idx_hbm, table_hbm, out_hbm, idx_v, bufs, isem, g0, g1, w0, w1):
    wid = lax.axis_index("s") * 2 + lax.axis_index("c")
    gsem = (g0, g1)
    wsem = (w0, w1)
    pltpu.async_copy(idx_hbm.at[wid], idx_v, isem).wait()

    def out_block(j):
      return out_hbm.at[pl.ds(wid * _NB + j * _CBATCH, _CBATCH)]

    def issue(j, b):
      buf = bufs.at[b]

      def grp_body(g, _):
        vec = idx_v[pl.ds(j * _CH + g * 16, 16)]
        for lane in range(16):
          r = vec[lane]
          pltpu.async_copy(
              table_hbm.at[pl.ds(r, 1)],
              buf.at[(g * 16 + lane) // _SEQ, pl.ds((g * 16 + lane) % _SEQ, 1)],
              gsem[b],
          )
        return 0

      lax.fori_loop(0, _CH // 16, grp_body, 0)
      vec = idx_v[pl.ds(j * _CH + (_CH // 16) * 16, 16)]
      for lane in range(_CH - (_CH // 16) * 16):
        i = (_CH // 16) * 16 + lane
        r = vec[lane]
        pltpu.async_copy(
            table_hbm.at[pl.ds(r, 1)],
            buf.at[i // _SEQ, pl.ds(i % _SEQ, 1)],
            gsem[b],
        )

    def drain_and_flush(j, b):
      buf = bufs.at[b]
      dst = out_block(j)
      pltpu.make_async_copy(dst, buf, gsem[b]).wait()
      pltpu.async_copy(buf, dst, wsem[b])

    issue(0, 0)
    issue(1, 1)

    def loop_body(j2, _):
      for b in range(2):
        j = j2 * 2 + b
        drain_and_flush(j, b)

        @pl.when(j + 2 < _NCH)
        def _():
          pltpu.make_async_copy(out_block(j), bufs.at[b], wsem[b]).wait()
          issue(j + 2, b)

      return 0

    lax.fori_loop(0, _NCH // 2, loop_body, 0)
    pltpu.make_async_copy(out_block(_NCH - 2), bufs.at[0], wsem[0]).wait()
    pltpu.make_async_copy(out_block(_NCH - 1), bufs.at[1], wsem[1]).wait()

  return k


_gather_kernel = _build()


def kernel(tok_idxs, embed):
  idx = tok_idxs.reshape(_NW, _BPW)
  return _gather_kernel(idx, embed)
